# baseline (speedup 1.0000x reference)
import functools

import jax
import jax.numpy as jnp
from jax.experimental import pallas as pl
from jax.experimental.pallas import tpu as pltpu


def _sigmoid(z):
    return 1.0 / (1.0 + jnp.exp(-z))


def _cbam_compute(nb, c, s, w, k, inv_s, inv_c, wsp_ref, wdt_ref, wut_ref,
                  x_flat_ref, o_flat_ref, pbuf_ref):
    f32 = jnp.float32
    x = x_flat_ref[...].reshape(nb, c, s)

    mx = jnp.max(x, axis=2)
    av = jnp.sum(x, axis=2) * inv_s
    pooled = jnp.concatenate([mx, av], axis=0)
    hid = jnp.maximum(
        jnp.dot(pooled, wdt_ref[...], preferred_element_type=f32), 0.0)
    z = jnp.dot(hid, wut_ref[...], preferred_element_type=f32)
    gate = _sigmoid(z[:nb] + z[nb:])

    x1 = x * gate[:, :, None]

    cmax = jnp.max(x1, axis=1)
    cmean = jnp.sum(x1, axis=1) * inv_c

    pbuf_ref[...] = jnp.zeros_like(pbuf_ref)
    pbuf_ref[0, :, 128:128 + s] = cmax
    pbuf_ref[1, :, 128:128 + s] = cmean

    col = jax.lax.broadcasted_iota(jnp.int32, (1, s), 1) % w
    half = k // 2
    acc = jnp.zeros((nb, s), f32)
    for dx in range(k):
        t0 = jnp.zeros((nb, s), f32)
        t1 = jnp.zeros((nb, s), f32)
        for dy in range(k):
            off = 128 + (dy - half) * w + (dx - half)
            t0 = t0 + wsp_ref[dy * k + dx] * pbuf_ref[0, :, off:off + s]
            t1 = t1 + wsp_ref[k * k + dy * k + dx] * pbuf_ref[1, :, off:off + s]
        m = (col + (dx - half) >= 0) & (col + (dx - half) < w)
        acc = acc + jnp.where(m, t0 + t1, 0.0)
    sgate = _sigmoid(acc)

    out = (x1 * sgate[:, None, :]).astype(o_flat_ref.dtype)
    o_flat_ref[...] = out.reshape(nb * c * s)


def _cbam_pipeline(nb, n_steps, c, s, w, k, inv_s, inv_c,
                   wsp_ref, x_hbm, wdt_ref, wut_ref,
                   o_hbm,
                   xbuf0, xbuf1, obuf0, obuf1, pbuf, in_sem, out_sem):
    compute = functools.partial(_cbam_compute, nb, c, s, w, k, inv_s, inv_c,
                                wsp_ref, wdt_ref, wut_ref)
    xbufs = (xbuf0, xbuf1)
    obufs = (obuf0, obuf1)
    blk = nb * c * s

    def start_in(slot, step):
        pltpu.make_async_copy(x_hbm.at[pl.ds(step * blk, blk)],
                              xbufs[slot], in_sem.at[slot]).start()

    def wait_in(slot):
        pltpu.make_async_copy(x_hbm.at[pl.ds(0, blk)],
                              xbufs[slot], in_sem.at[slot]).wait()

    def start_out(slot, step):
        pltpu.make_async_copy(obufs[slot],
                              o_hbm.at[pl.ds(step * blk, blk)],
                              out_sem.at[slot]).start()

    def wait_out(slot):
        pltpu.make_async_copy(obufs[slot],
                              o_hbm.at[pl.ds(0, blk)],
                              out_sem.at[slot]).wait()

    start_in(0, 0)
    if n_steps > 1:
        start_in(1, 1)
    for step in range(n_steps):
        slot = step % 2
        wait_in(slot)
        if step >= 2:
            wait_out(slot)
        compute(xbufs[slot], obufs[slot], pbuf)
        start_out(slot, step)
        if step + 2 < n_steps:
            start_in(slot, step + 2)
    wait_out((n_steps - 1) % 2)
    if n_steps > 1:
        wait_out(n_steps % 2)


def kernel(x_nchw, w_mlp_down, w_mlp_up, w_spatial):
    N, C, H, W = x_nchw.shape
    Cr = w_mlp_down.shape[0]
    K = w_spatial.shape[-1]
    S = H * W
    NB = 8
    f32 = jnp.float32

    x_flat = x_nchw.reshape(-1)
    wdt = w_mlp_down.reshape(Cr, C).T.astype(f32)
    wut = w_mlp_up.reshape(C, Cr).T.astype(f32)
    wsp = w_spatial.reshape(-1).astype(f32)

    kern = functools.partial(_cbam_pipeline, NB, N // NB, C, S, W, K,
                             1.0 / S, 1.0 / C)
    out_flat = pl.pallas_call(
        kern,
        out_shape=jax.ShapeDtypeStruct((N * C * S,), x_nchw.dtype),
        in_specs=[pl.BlockSpec(memory_space=pltpu.MemorySpace.SMEM),
                  pl.BlockSpec(memory_space=pl.ANY),
                  pl.BlockSpec(memory_space=pltpu.MemorySpace.VMEM),
                  pl.BlockSpec(memory_space=pltpu.MemorySpace.VMEM)],
        out_specs=pl.BlockSpec(memory_space=pl.ANY),
        scratch_shapes=[pltpu.VMEM((NB * C * S,), f32),
                        pltpu.VMEM((NB * C * S,), f32),
                        pltpu.VMEM((NB * C * S,), f32),
                        pltpu.VMEM((NB * C * S,), f32),
                        pltpu.VMEM((2, NB, S + 256), f32),
                        pltpu.SemaphoreType.DMA((2,)),
                        pltpu.SemaphoreType.DMA((2,))],
        compiler_params=pltpu.CompilerParams(
            vmem_limit_bytes=120 * 1024 * 1024),
    )(wsp, x_flat, wdt, wut)

    return out_flat.reshape(N, C, H, W)


# speedup vs baseline: 1.8690x; 1.8690x over previous
import functools

import jax
import jax.numpy as jnp
from jax.experimental import pallas as pl
from jax.experimental.pallas import tpu as pltpu


def _sigmoid(z):
    return 1.0 / (1.0 + jnp.exp(-z))


def _cbam_kernel(nb, c, w, k, inv_s, inv_c,
                 wsp_ref, x_ref, wdt_ref, wut_ref,
                 o_ref,
                 pbuf_ref):
    f32 = jnp.float32
    s = o_ref.shape[2]
    x = x_ref[...].reshape(nb, c, s)

    mx = jnp.max(x, axis=2)
    av = jnp.sum(x, axis=2) * inv_s
    pooled = jnp.concatenate([mx, av], axis=0)
    hid = jnp.maximum(
        jnp.dot(pooled, wdt_ref[...], preferred_element_type=f32), 0.0)
    z = jnp.dot(hid, wut_ref[...], preferred_element_type=f32)
    gate = _sigmoid(z[:nb] + z[nb:])

    x1 = x * gate[:, :, None]

    cmax = jnp.max(x1, axis=1)
    cmean = jnp.sum(x1, axis=1) * inv_c

    pbuf_ref[...] = jnp.zeros_like(pbuf_ref)
    pbuf_ref[0, :, 128:128 + s] = cmax
    pbuf_ref[1, :, 128:128 + s] = cmean

    col = jax.lax.broadcasted_iota(jnp.int32, (1, s), 1) % w
    half = k // 2
    acc = jnp.zeros((nb, s), f32)
    for dx in range(k):
        t = jnp.zeros((nb, s), f32)
        for ch in range(2):
            for dy in range(k):
                off = 128 + (dy - half) * w + (dx - half)
                t = t + wsp_ref[ch * k * k + dy * k + dx] * pbuf_ref[ch, :, off:off + s]
        m = (col + (dx - half) >= 0) & (col + (dx - half) < w)
        acc = acc + jnp.where(m, t, 0.0)
    sgate = _sigmoid(acc)

    o_ref[...] = (x1 * sgate[:, None, :]).astype(o_ref.dtype)


def kernel(x_nchw, w_mlp_down, w_mlp_up, w_spatial):
    N, C, H, W = x_nchw.shape
    Cr = w_mlp_down.shape[0]
    K = w_spatial.shape[-1]
    S = H * W
    NB = 2
    f32 = jnp.float32

    x3 = x_nchw.reshape(N * C, H, W)
    wdt = w_mlp_down.reshape(Cr, C).T.astype(f32)
    wut = w_mlp_up.reshape(C, Cr).T.astype(f32)
    wsp = w_spatial.reshape(-1).astype(f32)

    kern = functools.partial(_cbam_kernel, NB, C, W, K, 1.0 / S, 1.0 / C)
    out_ncs = pl.pallas_call(
        kern,
        out_shape=jax.ShapeDtypeStruct((N, C, S), x_nchw.dtype),
        grid=(N // NB,),
        in_specs=[pl.BlockSpec(memory_space=pltpu.MemorySpace.SMEM),
                  pl.BlockSpec((NB * C, H, W), lambda n: (n, 0, 0)),
                  pl.BlockSpec((C, Cr), lambda n: (0, 0)),
                  pl.BlockSpec((Cr, C), lambda n: (0, 0))],
        out_specs=pl.BlockSpec((NB, C, S), lambda n: (n, 0, 0)),
        scratch_shapes=[pltpu.VMEM((2, NB, S + 256), f32)],
        compiler_params=pltpu.CompilerParams(
            dimension_semantics=("parallel",),
            vmem_limit_bytes=96 * 1024 * 1024),
    )(wsp, x3, wdt, wut)

    return out_ncs.reshape(N, C, H, W)


# speedup vs baseline: 3.2617x; 1.7451x over previous
import functools

import jax
import jax.numpy as jnp
from jax.experimental import pallas as pl
from jax.experimental.pallas import tpu as pltpu


def _sigmoid(z):
    return 1.0 / (1.0 + jnp.exp(-z))


def _cbam_kernel(nb, w, k, inv_s, inv_c,
                 wsp_ref, x_ref, wdt_ref, wut_ref,
                 o_ref,
                 pbuf_ref):
    f32 = jnp.float32
    s = x_ref.shape[2]
    x = x_ref[...].astype(f32)

    mx = jnp.max(x, axis=2)
    av = jnp.sum(x, axis=2) * inv_s
    pooled = jnp.concatenate([mx, av], axis=0)
    hid = jnp.maximum(
        jnp.dot(pooled, wdt_ref[...], preferred_element_type=f32), 0.0)
    z = jnp.dot(hid, wut_ref[...], preferred_element_type=f32)
    gate = _sigmoid(z[:nb] + z[nb:])

    x1 = x * gate[:, :, None]

    cmax = jnp.max(x1, axis=1)
    cmean = jnp.sum(x1, axis=1) * inv_c

    pbuf_ref[...] = jnp.zeros_like(pbuf_ref)
    pbuf_ref[0, :, 128:128 + s] = cmax
    pbuf_ref[1, :, 128:128 + s] = cmean

    col = jax.lax.broadcasted_iota(jnp.int32, (1, s), 1) % w
    half = k // 2
    acc = jnp.zeros((nb, s), f32)
    for dx in range(k):
        t = jnp.zeros((nb, s), f32)
        for ch in range(2):
            for dy in range(k):
                off = 128 + (dy - half) * w + (dx - half)
                t = t + wsp_ref[ch * k * k + dy * k + dx] * pbuf_ref[ch, :, off:off + s]
        m = (col + (dx - half) >= 0) & (col + (dx - half) < w)
        acc = acc + jnp.where(m, t, 0.0)
    sgate = _sigmoid(acc)

    o_ref[...] = (x1 * sgate[:, None, :]).astype(o_ref.dtype)


def kernel(x_nchw, w_mlp_down, w_mlp_up, w_spatial):
    N, C, H, W = x_nchw.shape
    Cr = w_mlp_down.shape[0]
    K = w_spatial.shape[-1]
    S = H * W
    NB = 8
    f32 = jnp.float32

    x_ncs = x_nchw.reshape(N, C, S)
    wdt = w_mlp_down.reshape(Cr, C).T.astype(f32)
    wut = w_mlp_up.reshape(C, Cr).T.astype(f32)
    wsp = w_spatial.reshape(-1).astype(f32)

    kern = functools.partial(_cbam_kernel, NB, W, K, 1.0 / S, 1.0 / C)
    out_ncs = pl.pallas_call(
        kern,
        out_shape=jax.ShapeDtypeStruct((N, C, S), x_nchw.dtype),
        grid=(N // NB,),
        in_specs=[pl.BlockSpec(memory_space=pltpu.MemorySpace.SMEM),
                  pl.BlockSpec((NB, C, S), lambda n: (n, 0, 0)),
                  pl.BlockSpec((C, Cr), lambda n: (0, 0)),
                  pl.BlockSpec((Cr, C), lambda n: (0, 0))],
        out_specs=pl.BlockSpec((NB, C, S), lambda n: (n, 0, 0)),
        scratch_shapes=[pltpu.VMEM((2, NB, S + 256), f32)],
        compiler_params=pltpu.CompilerParams(
            dimension_semantics=("parallel",),
            vmem_limit_bytes=96 * 1024 * 1024),
    )(wsp, x_ncs, wdt, wut)

    return out_ncs.reshape(N, C, H, W)
